# baseline (speedup 1.0000x reference)
import functools

import jax
import jax.numpy as jnp
from jax import lax
from jax.experimental import pallas as pl
from jax.experimental.pallas import tpu as pltpu
from jax.experimental.pallas import tpu_sc as plsc

_NUM_CORES = 2
_NUM_SUBCORES = 16
_NUM_WORKERS = _NUM_CORES * _NUM_SUBCORES

_CHUNK = 128
_JP = 100


def _tr_body(x_ref, o_ref):
    o_ref[:, 0:64] = x_ref[...].T.astype(jnp.bfloat16)


def _transpose_w(w):
    d, v = w.shape
    blk = 8192
    return pl.pallas_call(
        _tr_body,
        grid=(pl.cdiv(v, blk),),
        in_specs=[pl.BlockSpec((d, blk), lambda i: (0, i))],
        out_specs=pl.BlockSpec((blk, 2 * d), lambda i: (i, 0)),
        out_shape=jax.ShapeDtypeStruct((v, 2 * d), jnp.bfloat16),
    )(w)


def _sc_gather(table, idx2d):
    v2, d = table.shape
    ni, nj = idx2d.shape
    ni_w = ni // _NUM_WORKERS

    mesh = plsc.VectorSubcoreMesh(
        core_axis_name="c",
        subcore_axis_name="s",
        num_cores=_NUM_CORES,
        num_subcores=_NUM_SUBCORES,
    )

    @functools.partial(
        pl.kernel,
        mesh=mesh,
        compiler_params=pltpu.CompilerParams(
            use_tc_tiling_on_sc=False, needs_layout_passes=False
        ),
        out_type=jax.ShapeDtypeStruct((nj, ni // 2, 2 * d), jnp.bfloat16),
        scratch_types=[
            pltpu.VMEM((ni_w, nj), jnp.int32),
            pltpu.VMEM((nj, ni_w), jnp.int32),
            pltpu.VMEM((ni_w, d), jnp.bfloat16),
            pltpu.VMEM((ni_w, d), jnp.bfloat16),
            pltpu.SemaphoreType.DMA,
        ],
    )
    def gather_kernel(
        table_hbm, idx_hbm, out_hbm, idx_vt, idx_v, g_v0, g_v1, sem
    ):
        wid = lax.axis_index("s") * _NUM_CORES + lax.axis_index("c")
        i0 = wid * ni_w
        half = ni // 2

        pltpu.sync_copy(idx_hbm.at[pl.ds(i0, ni_w)], idx_vt)
        for j in range(nj):
            cols = jnp.full((16,), j, jnp.int32)
            for k in range(ni_w // 16):
                rows = lax.iota(jnp.int32, 16) + (k * 16)
                idx_v[j, pl.ds(k * 16, 16)] = plsc.load_gather(
                    idx_vt, [rows, cols]
                )

        def fire(j, buf):
            pltpu.async_copy(table_hbm.at[idx_v.at[j]], buf, sem)

        def drain(j, buf):
            pltpu.make_async_copy(table_hbm.at[idx_v.at[j]], buf, sem).wait()

        def store(j, buf):
            pltpu.sync_copy(
                buf,
                out_hbm.at[
                    j,
                    pl.ds(lax.rem(i0, half), ni_w),
                    pl.ds((i0 // half) * d, d),
                ],
            )

        fire(0, g_v0)
        fire(1, g_v1)

        def body(h, carry):
            j0 = 2 * h
            drain(j0, g_v0)

            @pl.when(j0 + 2 < nj)
            def _():
                fire(j0 + 2, g_v0)

            store(j0, g_v0)
            drain(j0 + 1, g_v1)

            @pl.when(j0 + 3 < nj)
            def _():
                fire(j0 + 3, g_v1)

            store(j0 + 1, g_v1)
            return carry

        lax.fori_loop(0, nj // 2, body, 0)

    return gather_kernel(table, idx2d)


def _trg_body(x_ref, o_ref):
    for jj in range(8):
        for u in range(16):
            xf = x_ref[jj, pl.ds(128 * u, 128), :].astype(jnp.float32)
            xt = xf.T
            o_ref[:, jj, pl.ds(128 * u, 128)] = xt[0:64, :]
            o_ref[:, jj, pl.ds(2048 + 128 * u, 128)] = xt[64:128, :]


def _transpose_g(g3):
    nj, nm, _ = g3.shape
    ni = 2 * nm
    return pl.pallas_call(
        _trg_body,
        grid=(nj // 8,),
        in_specs=[pl.BlockSpec((8, nm, 128), lambda j: (j, 0, 0))],
        out_specs=pl.BlockSpec((64, 8, ni), lambda j: (0, j, 0)),
        out_shape=jax.ShapeDtypeStruct((64, nj, ni), jnp.float32),
    )(g3)


def kernel(vocab_idx, W):
    d, v = W.shape
    s0, s1 = vocab_idx.shape
    idx2d = vocab_idx.astype(jnp.int32) * 2
    t = _transpose_w(W).reshape(2 * v, d)
    g3 = _sc_gather(t, idx2d)
    p = _transpose_g(g3)
    return jnp.swapaxes(p, 1, 2)
